# baseline (speedup 1.0000x reference)
import functools

import jax
import jax.numpy as jnp
from jax import lax
from jax.experimental import pallas as pl
from jax.experimental.pallas import tpu as pltpu
from jax.experimental.pallas import tpu_sc as plsc

LABS = 30
SUBJ = 100
D = 128
D2 = 2 * D
B = 16384
NC = 2
NS = 16
NW = NC * NS
RPW = B // NW
CH = 32
NCH = RPW // CH
NBUF = 8

_mesh = plsc.VectorSubcoreMesh(core_axis_name="c", subcore_axis_name="s")


@functools.partial(
    pl.kernel,
    mesh=_mesh,
    out_type=jax.ShapeDtypeStruct((B, D2), jnp.float32),
    scratch_types=[
        pltpu.VMEM((RPW,), jnp.int32),
        pltpu.VMEM((NBUF, CH, D2), jnp.float32),
    ] + [pltpu.SemaphoreType.DMA] * 16,
)
def _adapter(idx1, table, out, idxv, rows, *sems):
    gsem = sems[:NBUF]
    wsem = sems[NBUF:]
    wid = lax.axis_index("s") * NC + lax.axis_index("c")
    base = wid * RPW
    pltpu.sync_copy(idx1.at[pl.ds(base, RPW)], idxv)
    gets = {}
    puts = {}
    for j in range(NBUF):
        gets[j] = pltpu.async_copy(
            table.at[idxv.at[pl.ds(j * CH, CH)]], rows.at[j], gsem[j])
    for j in range(NCH):
        b = j % NBUF
        if j >= NBUF:
            puts[j - NBUF].wait()
            gets[j] = pltpu.async_copy(
                table.at[idxv.at[pl.ds(j * CH, CH)]], rows.at[b], gsem[b])
        gets[j].wait()
        puts[j] = pltpu.async_copy(
            rows.at[b], out.at[pl.ds(base + j * CH, CH)], wsem[b])
    for j in range(NCH - NBUF, NCH):
        puts[j].wait()


def kernel(lab_ids, subject_ids, lab_table, subject_table):
    paired = jnp.concatenate([
        jnp.broadcast_to(lab_table[:, None, :], (LABS, SUBJ, D)),
        jnp.broadcast_to(subject_table[None, :, :], (LABS, SUBJ, D)),
    ], axis=-1).reshape(LABS * SUBJ, D2)
    idx = lab_ids * SUBJ + subject_ids
    return _adapter(idx, paired)


# speedup vs baseline: 1.1273x; 1.1273x over previous
import functools

import jax
import jax.numpy as jnp
from jax import lax
from jax.experimental import pallas as pl
from jax.experimental.pallas import tpu as pltpu
from jax.experimental.pallas import tpu_sc as plsc

LABS = 30
SUBJ = 100
D = 128
D2 = 2 * D
B = 16384
NC = 2
NS = 16
NW = NC * NS
RPW = B // NW
CH = 64
NCH = RPW // CH
NBUF = 6
L = 16

_mesh = plsc.VectorSubcoreMesh(core_axis_name="c", subcore_axis_name="s")


@functools.partial(
    pl.kernel,
    mesh=_mesh,
    out_type=jax.ShapeDtypeStruct((B, D2), jnp.float32),
    scratch_types=[
        pltpu.VMEM((RPW,), jnp.int32),
        pltpu.VMEM((RPW,), jnp.int32),
        pltpu.VMEM((RPW,), jnp.int32),
        pltpu.VMEM((NBUF, CH, D2), jnp.float32),
    ] + [pltpu.SemaphoreType.DMA] * 16,
)
def _adapter(lab_ids, sub_ids, table, out, lidv, sidv, idxv, rows, *sems):
    gsem = sems[:NBUF]
    wsem = sems[NBUF:]
    wid = lax.axis_index("s") * NC + lax.axis_index("c")
    base = wid * RPW
    pltpu.sync_copy(lab_ids.at[pl.ds(base, RPW)], lidv)
    pltpu.sync_copy(sub_ids.at[pl.ds(base, RPW)], sidv)
    for g in range(RPW // L):
        sl = pl.ds(g * L, L)
        idxv[sl] = lidv[sl] * SUBJ + sidv[sl]
    gets = {}
    puts = {}
    for j in range(NBUF):
        gets[j] = pltpu.async_copy(
            table.at[idxv.at[pl.ds(j * CH, CH)]], rows.at[j], gsem[j])
    for j in range(NCH):
        b = j % NBUF
        if j >= NBUF:
            puts[j - NBUF].wait()
            gets[j] = pltpu.async_copy(
                table.at[idxv.at[pl.ds(j * CH, CH)]], rows.at[b], gsem[b])
        gets[j].wait()
        puts[j] = pltpu.async_copy(
            rows.at[b], out.at[pl.ds(base + j * CH, CH)], wsem[b])
    for j in range(NCH - NBUF, NCH):
        puts[j].wait()


def kernel(lab_ids, subject_ids, lab_table, subject_table):
    paired = jnp.concatenate([
        jnp.broadcast_to(lab_table[:, None, :], (LABS, SUBJ, D)),
        jnp.broadcast_to(subject_table[None, :, :], (LABS, SUBJ, D)),
    ], axis=-1).reshape(LABS * SUBJ, D2)
    return _adapter(lab_ids, subject_ids, paired)
